# baseline (speedup 1.0000x reference)
import functools

import jax
import jax.numpy as jnp
from jax import lax
from jax.experimental import pallas as pl
from jax.experimental.pallas import tpu as pltpu
from jax.experimental.pallas import tpu_sc as plsc

B = 500
N_PER = 100
N = B * N_PER
DEG = 16
E = N * DEG
EPG = N_PER * DEG
APG = N_PER * N_PER
IN_DIM = 3
HID = 64
OUT_DIM = 64
FLAT = N_PER * OUT_DIM

NC = 2
NS = 16
NW = NC * NS
LANES = 16


def _make_adj_body(goff, nb):
    gmax = (nb + NW - 1) // NW
    grem = nb - NW * (gmax - 1)

    def _adj_body(src_hbm, dst_hbm, a_hbm, src_v, dst_v, acc0, acc1, sem0, sem1):
        cid = lax.axis_index("c")
        sid = lax.axis_index("s")
        wid = sid * NC + cid
        start = wid * (gmax - 1) + jnp.minimum(wid, grem)
        has_last = wid < grem

        ones = jnp.ones((LANES,), jnp.float32)
        zeros = jnp.zeros((LANES,), jnp.float32)
        accs = (acc0, acc1)
        sems = (sem0, sem1)

        def out_copy(t, acc, sem):
            g = start + t
            return pltpu.make_async_copy(acc, a_hbm.at[pl.ds(g * APG, APG)], sem)

        def build_graph(t, acc):
            g = start + t

            def zbody(i, _):
                acc[pl.ds(i * LANES, LANES)] = zeros
                return 0
            lax.fori_loop(0, APG // LANES, zbody, 0, unroll=8)

            pltpu.sync_copy(src_hbm.at[pl.ds(g * EPG, EPG)], src_v)
            pltpu.sync_copy(dst_hbm.at[pl.ds(g * EPG, EPG)], dst_v)

            shift = (g + goff) * N_PER * (N_PER + 1)

            def ebody(j, _):
                sv = src_v[pl.ds(j * LANES, LANES)]
                dv = dst_v[pl.ds(j * LANES, LANES)]
                idx = dv * N_PER + sv - shift
                plsc.addupdate_scatter(acc, [idx], ones)
                return 0
            lax.fori_loop(0, EPG // LANES, ebody, 0, unroll=4)

        for t in range(gmax - 1):
            p = t % 2
            if t >= 2:
                out_copy(t - 2, accs[p], sems[p]).wait()
            build_graph(t, accs[p])
            out_copy(t, accs[p], sems[p]).start()

        t_last = gmax - 1
        p_last = t_last % 2

        @pl.when(has_last)
        def _():
            if t_last >= 2:
                out_copy(t_last - 2, accs[p_last], sems[p_last]).wait()
            build_graph(t_last, accs[p_last])
            out_copy(t_last, accs[p_last], sems[p_last]).start()

        if gmax >= 2:
            out_copy(gmax - 2, accs[(gmax - 2) % 2], sems[(gmax - 2) % 2]).wait()

        @pl.when(has_last)
        def _():
            out_copy(t_last, accs[p_last], sems[p_last]).wait()

        if t_last >= 2:
            @pl.when(jnp.logical_not(has_last))
            def _():
                out_copy(t_last - 2, accs[p_last], sems[p_last]).wait()

    return _adj_body


def _build_adj(src_chunk, dst_chunk, goff, nb):
    mesh = plsc.VectorSubcoreMesh(core_axis_name="c", subcore_axis_name="s")
    kern = pl.kernel(
        _make_adj_body(goff, nb),
        out_type=jax.ShapeDtypeStruct((nb * APG,), jnp.float32),
        mesh=mesh,
        scratch_types=[
            pltpu.VMEM((EPG,), jnp.int32),
            pltpu.VMEM((EPG,), jnp.int32),
            pltpu.VMEM((APG,), jnp.float32),
            pltpu.VMEM((APG,), jnp.float32),
            pltpu.SemaphoreType.DMA,
            pltpu.SemaphoreType.DMA,
        ],
        compiler_params=pltpu.CompilerParams(needs_layout_passes=False),
    )
    return kern(src_chunk, dst_chunk)


GB = 20


def _gin_body(a_ref, x_ref,
              w01_ref, b01_ref, w02_ref, b02_ref,
              w11_ref, b11_ref, w12_ref, b12_ref,
              w21_ref, b21_ref, w22_ref, b22_ref,
              wp_ref, bp_ref, hp_ref, agg_ref):
    f32 = jnp.float32

    def dot(a, b):
        return jnp.dot(a, b, preferred_element_type=f32)

    def spread_agg(h):
        for g in range(GB):
            sl = pl.ds(g * N_PER, N_PER)
            agg_ref[sl, :] = dot(a_ref[sl, :], h[g * N_PER:(g + 1) * N_PER, :])
        return agg_ref[...]

    u = dot(x_ref[...], w01_ref[...])
    z = jnp.maximum(u + spread_agg(u) + b01_ref[...], 0.0)
    h = jnp.maximum(dot(z, w02_ref[...]) + b02_ref[...], 0.0)
    for (w1, b1, w2, b2) in ((w11_ref, b11_ref, w12_ref, b12_ref),
                             (w21_ref, b21_ref, w22_ref, b22_ref)):
        z = jnp.maximum(dot(h + spread_agg(h), w1[...]) + b1[...], 0.0)
        h = jnp.maximum(dot(z, w2[...]) + b2[...], 0.0)
    hp_ref[...] = dot(h, wp_ref[...]) + bp_ref[...]


def _gin(a2d, x, p, nb):
    grid = (nb // GB,)
    row_blk = GB * N_PER

    def wspec(shape):
        return pl.BlockSpec(shape, lambda i: (0, 0))

    in_specs = [
        pl.BlockSpec((row_blk, N_PER), lambda i: (i, 0)),
        pl.BlockSpec((row_blk, IN_DIM), lambda i: (i, 0)),
        wspec((IN_DIM, HID)), wspec((1, HID)),
        wspec((HID, HID)), wspec((1, HID)),
        wspec((HID, HID)), wspec((1, HID)),
        wspec((HID, HID)), wspec((1, HID)),
        wspec((HID, HID)), wspec((1, HID)),
        wspec((HID, HID)), wspec((1, HID)),
        wspec((HID, OUT_DIM)), wspec((1, OUT_DIM)),
    ]
    out_spec = pl.BlockSpec((row_blk, OUT_DIM), lambda i: (i, 0))
    return pl.pallas_call(
        _gin_body,
        grid=grid,
        in_specs=in_specs,
        out_specs=out_spec,
        out_shape=jax.ShapeDtypeStruct((nb * N_PER, OUT_DIM), jnp.float32),
        scratch_shapes=[pltpu.VMEM((row_blk, HID), jnp.float32)],
        compiler_params=pltpu.CompilerParams(
            dimension_semantics=("arbitrary",)),
    )(a2d, x,
      p["gin0_W1"], p["gin0_b1"].reshape(1, HID),
      p["gin0_W2"], p["gin0_b2"].reshape(1, HID),
      p["gin1_W1"], p["gin1_b1"].reshape(1, HID),
      p["gin1_W2"], p["gin1_b2"].reshape(1, HID),
      p["gin2_W1"], p["gin2_b1"].reshape(1, HID),
      p["gin2_W2"], p["gin2_b2"].reshape(1, HID),
      p["proj_W"], p["proj_b"].reshape(1, OUT_DIM))


GBM = 500
MLP_HID = 64
BOTTLE = 128
LATENT = 64


def _mlp_body(f_ref, w1_ref, b1_ref, w2_ref, b2_ref,
              wm_ref, bm_ref, ws_ref, bs_ref, mean_ref, std_ref):
    f32 = jnp.float32

    def dot(a, b):
        return jnp.dot(a, b, preferred_element_type=f32)

    f = jnp.maximum(dot(f_ref[...], w1_ref[...]) + b1_ref[...], 0.0)
    f = jnp.maximum(dot(f, w2_ref[...]) + b2_ref[...], 0.0)
    mean_ref[...] = dot(f, wm_ref[...]) + bm_ref[...]
    s = dot(f, ws_ref[...]) + bs_ref[...]
    std_ref[...] = jnp.maximum(s, 0.0) + jnp.log(1.0 + jnp.exp(-jnp.abs(s)))


def _mlp(feat, p, nb):
    grid = (1,)

    def wspec(shape):
        return pl.BlockSpec(shape, lambda i: (0, 0))

    in_specs = [
        pl.BlockSpec((nb, FLAT), lambda i: (i, 0)),
        wspec((FLAT, MLP_HID)), wspec((1, MLP_HID)),
        wspec((MLP_HID, BOTTLE)), wspec((1, BOTTLE)),
        wspec((BOTTLE, LATENT)), wspec((1, LATENT)),
        wspec((BOTTLE, LATENT)), wspec((1, LATENT)),
    ]
    out_specs = [
        pl.BlockSpec((nb, LATENT), lambda i: (i, 0)),
        pl.BlockSpec((nb, LATENT), lambda i: (i, 0)),
    ]
    return pl.pallas_call(
        _mlp_body,
        grid=grid,
        in_specs=in_specs,
        out_specs=out_specs,
        out_shape=[jax.ShapeDtypeStruct((nb, LATENT), jnp.float32),
                   jax.ShapeDtypeStruct((nb, LATENT), jnp.float32)],
        compiler_params=pltpu.CompilerParams(
            dimension_semantics=("arbitrary",)),
    )(feat,
      p["mlp_W1"], p["mlp_b1"].reshape(1, MLP_HID),
      p["mlp_W2"], p["mlp_b2"].reshape(1, BOTTLE),
      p["mean_W"], p["mean_b"].reshape(1, LATENT),
      p["std_W"], p["std_b"].reshape(1, LATENT))


CHUNKS = ((0, 100), (100, 200), (300, 200))


def kernel(x, params, edge_index):
    src = edge_index[0]
    dst = edge_index[1]
    means, stds = [], []
    for goff, nb in CHUNKS:
        src_c = src[goff * EPG:(goff + nb) * EPG]
        dst_c = dst[goff * EPG:(goff + nb) * EPG]
        a_flat = _build_adj(src_c, dst_c, goff, nb)
        a2d = a_flat.reshape(nb * N_PER, N_PER)
        x_c = x[goff * N_PER:(goff + nb) * N_PER]
        hp = _gin(a2d, x_c, params, nb)
        feat = hp.reshape(nb, FLAT)
        mean, std = _mlp(feat, params, nb)
        means.append(mean)
        stds.append(std)
    return jnp.concatenate(means, axis=0), jnp.concatenate(stds, axis=0)


# speedup vs baseline: 1.2402x; 1.2402x over previous
import jax
import jax.numpy as jnp
from jax import lax
from jax.experimental import pallas as pl
from jax.experimental.pallas import tpu as pltpu
from jax.experimental.pallas import tpu_sc as plsc

B = 500
N_PER = 100
N = B * N_PER
NP_PAD = 128
DEG = 16
E = N * DEG
EPG = N_PER * DEG
IN_DIM = 3
HID = 64
OUT_DIM = 64
FLAT = N_PER * OUT_DIM
FLAT_PAD = N_PER * NP_PAD
MLP_HID = 64
BOTTLE = 128
LATENT = 64

NC = 2
NS = 16
NW = NC * NS
LANES = 16

CHUNKS = ((0, 100), (100, 200), (300, 200))


def _make_adj_body(goff, nb):
    gmax = (nb + NW - 1) // NW
    grem = nb - NW * (gmax - 1)

    def _adj_body(src_hbm, dst_hbm, a_hbm, src_v, dst_v, acc0, acc1, sem0, sem1):
        cid = lax.axis_index("c")
        sid = lax.axis_index("s")
        wid = sid * NC + cid
        start = wid * (gmax - 1) + jnp.minimum(wid, grem)
        has_last = wid < grem

        ones = jnp.ones((LANES,), jnp.float32)
        zeros = jnp.zeros((LANES,), jnp.float32)
        accs = (acc0, acc1)
        sems = (sem0, sem1)

        APGP = N_PER * NP_PAD

        def out_copy(t, acc, sem):
            g = start + t
            return pltpu.make_async_copy(
                acc, a_hbm.at[pl.ds(g * APGP, APGP)], sem)

        def build_graph(t, acc):
            g = start + t
            gg = g + goff

            def zbody(i, _):
                acc[pl.ds(i * LANES, LANES)] = zeros
                return 0
            lax.fori_loop(0, N_PER * NP_PAD // LANES, zbody, 0, unroll=8)

            pltpu.sync_copy(src_hbm.at[pl.ds(gg * EPG, EPG)], src_v)
            pltpu.sync_copy(dst_hbm.at[pl.ds(gg * EPG, EPG)], dst_v)

            shift = gg * N_PER * (NP_PAD + 1)

            def ebody(j, _):
                sv = src_v[pl.ds(j * LANES, LANES)]
                dv = dst_v[pl.ds(j * LANES, LANES)]
                idx = dv * NP_PAD + sv - shift
                plsc.addupdate_scatter(acc, [idx], ones)
                return 0
            lax.fori_loop(0, EPG // LANES, ebody, 0, unroll=4)

        for t in range(gmax - 1):
            p = t % 2
            if t >= 2:
                out_copy(t - 2, accs[p], sems[p]).wait()
            build_graph(t, accs[p])
            out_copy(t, accs[p], sems[p]).start()

        t_last = gmax - 1
        p_last = t_last % 2

        @pl.when(has_last)
        def _():
            if t_last >= 2:
                out_copy(t_last - 2, accs[p_last], sems[p_last]).wait()
            build_graph(t_last, accs[p_last])
            out_copy(t_last, accs[p_last], sems[p_last]).start()

        if gmax >= 2:
            out_copy(gmax - 2, accs[(gmax - 2) % 2], sems[(gmax - 2) % 2]).wait()

        @pl.when(has_last)
        def _():
            out_copy(t_last, accs[p_last], sems[p_last]).wait()

        if t_last >= 2:
            @pl.when(jnp.logical_not(has_last))
            def _():
                out_copy(t_last - 2, accs[p_last], sems[p_last]).wait()

    return _adj_body


def _build_adj(src, dst, goff, nb):
    mesh = plsc.VectorSubcoreMesh(core_axis_name="c", subcore_axis_name="s")
    kern = pl.kernel(
        _make_adj_body(goff, nb),
        out_type=jax.ShapeDtypeStruct((nb * N_PER * NP_PAD,), jnp.float32),
        mesh=mesh,
        scratch_types=[
            pltpu.VMEM((EPG,), jnp.int32),
            pltpu.VMEM((EPG,), jnp.int32),
            pltpu.VMEM((N_PER * NP_PAD,), jnp.float32),
            pltpu.VMEM((N_PER * NP_PAD,), jnp.float32),
            pltpu.SemaphoreType.DMA,
            pltpu.SemaphoreType.DMA,
        ],
        compiler_params=pltpu.CompilerParams(needs_layout_passes=False),
    )
    return kern(src, dst)


GB = 20


def _gin_body(a_ref, x_ref,
              w01_ref, b01_ref, w02_ref, b02_ref,
              w11_ref, b11_ref, w12_ref, b12_ref,
              w21_ref, b21_ref, w22_ref, b22_ref,
              wp_ref, bp_ref, hp_ref, agg_ref):
    f32 = jnp.float32
    pad = jnp.zeros((NP_PAD - N_PER, HID), f32)

    def dot(a, b):
        return jnp.dot(a, b, preferred_element_type=f32)

    def spread_agg(h):
        for g in range(GB):
            sl = pl.ds(g * N_PER, N_PER)
            hg = jnp.concatenate([h[g * N_PER:(g + 1) * N_PER, :], pad], axis=0)
            agg_ref[sl, :] = dot(a_ref[sl, :], hg)
        return agg_ref[...]

    u = dot(x_ref[...], w01_ref[...])
    z = jnp.maximum(u + spread_agg(u) + b01_ref[...], 0.0)
    h = jnp.maximum(dot(z, w02_ref[...]) + b02_ref[...], 0.0)
    for (w1, b1, w2, b2) in ((w11_ref, b11_ref, w12_ref, b12_ref),
                             (w21_ref, b21_ref, w22_ref, b22_ref)):
        z = jnp.maximum(dot(h + spread_agg(h), w1[...]) + b1[...], 0.0)
        h = jnp.maximum(dot(z, w2[...]) + b2[...], 0.0)
    hp = dot(h, wp_ref[...]) + bp_ref[...]
    hp_ref[...] = jnp.concatenate(
        [hp, jnp.zeros((GB * N_PER, NP_PAD - OUT_DIM), f32)], axis=1)


def _gin(a2d, x, p, goff, nb):
    grid = (nb // GB,)
    row_blk = GB * N_PER
    xoff = goff // GB

    def wspec(shape):
        return pl.BlockSpec(shape, lambda i: (0, 0))

    in_specs = [
        pl.BlockSpec((row_blk, NP_PAD), lambda i: (i, 0)),
        pl.BlockSpec((row_blk, IN_DIM), lambda i: (i + xoff, 0)),
        wspec((IN_DIM, HID)), wspec((1, HID)),
        wspec((HID, HID)), wspec((1, HID)),
        wspec((HID, HID)), wspec((1, HID)),
        wspec((HID, HID)), wspec((1, HID)),
        wspec((HID, HID)), wspec((1, HID)),
        wspec((HID, HID)), wspec((1, HID)),
        wspec((HID, OUT_DIM)), wspec((1, OUT_DIM)),
    ]
    out_spec = pl.BlockSpec((row_blk, NP_PAD), lambda i: (i, 0))
    return pl.pallas_call(
        _gin_body,
        grid=grid,
        in_specs=in_specs,
        out_specs=out_spec,
        out_shape=jax.ShapeDtypeStruct((nb * N_PER, NP_PAD), jnp.float32),
        scratch_shapes=[pltpu.VMEM((row_blk, HID), jnp.float32)],
        compiler_params=pltpu.CompilerParams(
            dimension_semantics=("arbitrary",)),
    )(a2d, x,
      p["gin0_W1"], p["gin0_b1"].reshape(1, HID),
      p["gin0_W2"], p["gin0_b2"].reshape(1, HID),
      p["gin1_W1"], p["gin1_b1"].reshape(1, HID),
      p["gin1_W2"], p["gin1_b2"].reshape(1, HID),
      p["gin2_W1"], p["gin2_b1"].reshape(1, HID),
      p["gin2_W2"], p["gin2_b2"].reshape(1, HID),
      p["proj_W"], p["proj_b"].reshape(1, OUT_DIM))


def _mlp_body(f_ref, w1_ref, b1_ref, w2_ref, b2_ref,
              wm_ref, bm_ref, ws_ref, bs_ref, mean_ref, std_ref):
    f32 = jnp.float32

    def dot(a, b):
        return jnp.dot(a, b, preferred_element_type=f32)

    f = jnp.maximum(dot(f_ref[...], w1_ref[...]) + b1_ref[...], 0.0)
    f = jnp.maximum(dot(f, w2_ref[...]) + b2_ref[...], 0.0)
    mean_ref[...] = dot(f, wm_ref[...]) + bm_ref[...]
    s = dot(f, ws_ref[...]) + bs_ref[...]
    std_ref[...] = jnp.maximum(s, 0.0) + jnp.log(1.0 + jnp.exp(-jnp.abs(s)))


def _mlp(feat, w1p, p, nb):
    def wspec(shape):
        return pl.BlockSpec(shape, lambda i: (0, 0))

    in_specs = [
        pl.BlockSpec((nb, FLAT_PAD), lambda i: (i, 0)),
        wspec((FLAT_PAD, MLP_HID)), wspec((1, MLP_HID)),
        wspec((MLP_HID, BOTTLE)), wspec((1, BOTTLE)),
        wspec((BOTTLE, LATENT)), wspec((1, LATENT)),
        wspec((BOTTLE, LATENT)), wspec((1, LATENT)),
    ]
    out_specs = [
        pl.BlockSpec((nb, LATENT), lambda i: (i, 0)),
        pl.BlockSpec((nb, LATENT), lambda i: (i, 0)),
    ]
    return pl.pallas_call(
        _mlp_body,
        grid=(1,),
        in_specs=in_specs,
        out_specs=out_specs,
        out_shape=[jax.ShapeDtypeStruct((nb, LATENT), jnp.float32),
                   jax.ShapeDtypeStruct((nb, LATENT), jnp.float32)],
        compiler_params=pltpu.CompilerParams(
            dimension_semantics=("arbitrary",)),
    )(feat,
      w1p, p["mlp_b1"].reshape(1, MLP_HID),
      p["mlp_W2"], p["mlp_b2"].reshape(1, BOTTLE),
      p["mean_W"], p["mean_b"].reshape(1, LATENT),
      p["std_W"], p["std_b"].reshape(1, LATENT))


def kernel(x, params, edge_index):
    src = edge_index[0]
    dst = edge_index[1]
    w1p = jnp.pad(params["mlp_W1"].reshape(N_PER, OUT_DIM, MLP_HID),
                  ((0, 0), (0, NP_PAD - OUT_DIM), (0, 0))
                  ).reshape(FLAT_PAD, MLP_HID)
    means, stds = [], []
    for goff, nb in CHUNKS:
        a_flat = _build_adj(src, dst, goff, nb)
        a2d = a_flat.reshape(nb * N_PER, NP_PAD)
        hp = _gin(a2d, x, params, goff, nb)
        feat = hp.reshape(nb, FLAT_PAD)
        mean, std = _mlp(feat, w1p, params, nb)
        means.append(mean)
        stds.append(std)
    return jnp.concatenate(means, axis=0), jnp.concatenate(stds, axis=0)
